# baseline (speedup 1.0000x reference)
import jax
import jax.numpy as jnp
from jax import lax
from jax.experimental import pallas as pl
from jax.experimental.pallas import tpu as pltpu
from jax.experimental.pallas import tpu_sc as plsc

_VOCAB = 1000
_B = 1024
_L = 50
_VP = 1024
_NC, _NS = 2, 16
_NW = _NC * _NS
_NSEG = _B // 16
_SPLIT = 29


def kernel(x, table):
    xt = jnp.transpose(x).reshape(_L * _B).astype(jnp.int32)
    tblT = jnp.pad(
        jnp.transpose(table), ((0, _VP - _VOCAB), (0, _VP - _VOCAB))
    ).reshape(_VP * _VP)

    mesh = plsc.VectorSubcoreMesh(core_axis_name="c", subcore_axis_name="s")

    @pl.kernel(
        out_type=jax.ShapeDtypeStruct((_L, _VOCAB, _B), table.dtype),
        mesh=mesh,
        compiler_params=pltpu.CompilerParams(
            use_tc_tiling_on_sc=True, needs_layout_passes=False),
        scratch_types=[
            pltpu.VMEM((32 * _VP,), jnp.float32),
            pltpu.VMEM((_L * _B,), jnp.int32),
            pltpu.VMEM((8, _B), jnp.float32),
            pltpu.VMEM((8, _B), jnp.float32),
            pltpu.SemaphoreType.DMA,
            pltpu.SemaphoreType.DMA,
        ],
    )
    def lookup_kernel(tbl_hbm, xt_hbm, q_hbm, tbl_v, idx_v, ob0, ob1, ws0, ws1):
        obufs = (ob0, ob1)
        wsems = (ws0, ws1)
        wid = lax.axis_index("s") * _NC + lax.axis_index("c")
        small = wid >= _SPLIT
        nvt = jnp.where(small, 3, 4)
        vt0 = jnp.where(small, 4 * _SPLIT + 3 * (wid - _SPLIT), 4 * wid)
        v0 = vt0 * 8

        pltpu.sync_copy(tbl_hbm.at[pl.ds(v0 * _VP, 32 * _VP)], tbl_v)
        pltpu.sync_copy(xt_hbm, idx_v)

        total = _L * nvt

        def compute(l, j, ob):
            rows = [tbl_v.at[pl.ds((j * 8 + vi) * _VP, _VP)] for vi in range(8)]

            @pl.loop(0, _NSEG, step=4)
            def _(s):
                for t in range(4):
                    idx16 = idx_v[pl.ds(l * _B + (s + t) * 16, 16)]
                    for vi in range(8):
                        vals = plsc.load_gather(rows[vi], [idx16])
                        ob[vi, pl.ds((s + t) * 16, 16)] = vals

        def write_desc(l, j, p):
            return pltpu.make_async_copy(
                obufs[p], q_hbm.at[l].at[pl.ds(v0 + j * 8, 8)], wsems[p])

        @pl.loop(0, total, step=2)
        def _(u):
            for p in range(2):
                uu = u + p
                l = lax.div(uu, nvt)
                j = uu - l * nvt

                @pl.when(uu >= 2)
                def _():
                    write_desc(l, j, p).wait()

                compute(l, j, obufs[p])
                write_desc(l, j, p).start()

        for p in range(2):
            lastu = total - 2 + p
            ll = lax.div(lastu, nvt)
            write_desc(ll, lastu - ll * nvt, p).wait()

    q = lookup_kernel(tblT, xt)
    return jnp.transpose(q, (2, 0, 1))
